# baseline (speedup 1.0000x reference)
import functools

import jax
import jax.numpy as jnp
from jax.experimental import pallas as pl
from jax.experimental.pallas import tpu as pltpu

D_MODEL_ = 1024
D_HID_ = 4096
E_ = 8
TOPK_ = 2

T_ROWS = 256


def _router_body(x_ref, gw_ref, idx_ref, w_ref, rank_ref, counts_ref):
    N = x_ref.shape[0]
    logits = jnp.dot(x_ref[...], gw_ref[...], preferred_element_type=jnp.float32)
    m = jnp.max(logits, axis=-1, keepdims=True)
    ex = jnp.exp(logits - m)
    probs = ex / jnp.sum(ex, axis=-1, keepdims=True)

    ncols = probs.shape[-1]
    iota = jax.lax.broadcasted_iota(jnp.int32, probs.shape, 1)
    big = jnp.int32(ncols)

    m1 = jnp.max(probs, axis=-1, keepdims=True)
    i1 = jnp.min(jnp.where(probs == m1, iota, big), axis=-1, keepdims=True)
    mask1 = iota == i1
    probs2 = jnp.where(mask1, -jnp.inf, probs)
    m2 = jnp.max(probs2, axis=-1, keepdims=True)
    i2 = jnp.min(jnp.where(probs2 == m2, iota, big), axis=-1, keepdims=True)
    mask2 = iota == i2

    denom = m1 + m2
    idx_ref[0] = i1
    idx_ref[1] = i2
    w_ref[0] = m1 / denom
    w_ref[1] = m2 / denom

    BK = 128
    NB = N // BK
    oh = mask1.astype(jnp.float32) + mask2.astype(jnp.float32)
    oh3 = oh.reshape(NB, BK, ncols)
    li = jax.lax.broadcasted_iota(jnp.int32, (BK, BK), 0)
    lj = jax.lax.broadcasted_iota(jnp.int32, (BK, BK), 1)
    ltri = (lj <= li).astype(jnp.float32)
    intra = [
        jnp.dot(ltri, oh3[k], preferred_element_type=jnp.float32)
        for k in range(NB)
    ]
    bs = jnp.concatenate([intra[k][BK - 1 : BK, :] for k in range(NB)], axis=0)
    si = jax.lax.broadcasted_iota(jnp.int32, (NB, NB), 0)
    sj = jax.lax.broadcasted_iota(jnp.int32, (NB, NB), 1)
    stri = (sj < si).astype(jnp.float32)
    off = jnp.dot(stri, bs, preferred_element_type=jnp.float32)
    cum_inc = jnp.concatenate(intra, axis=0) + jnp.repeat(off, BK, axis=0)
    before = cum_inc - oh
    rank_ref[0] = jnp.sum(
        before * mask1.astype(jnp.float32), axis=-1, keepdims=True
    ).astype(jnp.int32)
    rank_ref[1] = jnp.sum(
        before * mask2.astype(jnp.float32), axis=-1, keepdims=True
    ).astype(jnp.int32)
    counts_ref[...] = (off[NB - 1 : NB, :] + bs[NB - 1 : NB, :])


def _up_body(texp_ref, tvalid_ref, hslot_ref, pos_ref, x_ref, w1_ref, b1_ref,
             h_ref, xs_ref):
    t = pl.program_id(0)
    s = pl.program_id(1)
    N = x_ref.shape[0]

    @pl.when(jnp.logical_and(tvalid_ref[t] > 0, s == 0))
    def _():
        base = t * T_ROWS
        p0 = pos_ref[0, :, 0][None, :]
        p1 = pos_ref[1, :, 0][None, :]
        rowi = jax.lax.broadcasted_iota(jnp.int32, (T_ROWS, N), 0) + base
        g = ((rowi == p0) | (rowi == p1)).astype(jnp.float32)
        xs_ref[...] = jnp.dot(g, x_ref[...], preferred_element_type=jnp.float32)

    @pl.when(tvalid_ref[t] > 0)
    def _():
        h = jnp.dot(xs_ref[...], w1_ref[0], preferred_element_type=jnp.float32)
        h = h + b1_ref[0]
        h = 0.5 * h * (1.0 + jax.lax.erf(h * (2.0 ** -0.5)))
        h_ref[0] = h.astype(jnp.bfloat16)


def _down_body(texp_ref, tvalid_ref, hslot_ref, pos_ref, w_ref, h_ref, w2_ref,
               b2_ref, out_ref, ys_ref):
    t = pl.program_id(0)
    N = out_ref.shape[0]

    s = pl.program_id(1)

    @pl.when(jnp.logical_and(t == 0, s == 0))
    def _():
        out_ref[...] = jnp.zeros_like(out_ref)

    @pl.when(tvalid_ref[t] > 0)
    def _():
        h = h_ref[0].astype(jnp.float32)
        part = jnp.dot(h, w2_ref[0], preferred_element_type=jnp.float32)

        @pl.when(s == 0)
        def _():
            ys_ref[...] = part

        @pl.when(s == 1)
        def _():
            ys = ys_ref[...] + part + b2_ref[0]
            base = t * T_ROWS
            p0 = pos_ref[0, :, 0][:, None]
            p1 = pos_ref[1, :, 0][:, None]
            w0 = w_ref[0, :, 0][:, None]
            w1v = w_ref[1, :, 0][:, None]
            coli = jax.lax.broadcasted_iota(jnp.int32, (N, T_ROWS), 1) + base
            pw = jnp.where(coli == p0, w0, 0.0) + jnp.where(coli == p1, w1v, 0.0)
            out_ref[...] += jnp.dot(pw, ys, preferred_element_type=jnp.float32)


@jax.jit
def kernel(x, gate_w, w1, w2, b1, b2):
    B, T, D = x.shape
    N = B * T
    M = N * TOPK_
    NT = M // T_ROWS + E_
    x_flat = x.reshape(N, D)

    idx_out, w_out, rank_out, counts_out = pl.pallas_call(
        _router_body,
        out_shape=(
            jax.ShapeDtypeStruct((TOPK_, N, 1), jnp.int32),
            jax.ShapeDtypeStruct((TOPK_, N, 1), jnp.float32),
            jax.ShapeDtypeStruct((TOPK_, N, 1), jnp.int32),
            jax.ShapeDtypeStruct((1, E_), jnp.float32),
        ),
    )(x_flat, gate_w)

    counts = counts_out[0].astype(jnp.int32)
    num_tiles_e = -(-counts // T_ROWS)
    cum_tiles = jnp.cumsum(num_tiles_e)
    tile_start = cum_tiles - num_tiles_e
    e0 = idx_out[0, :, 0]
    e1 = idx_out[1, :, 0]
    pos0 = tile_start[e0] * T_ROWS + rank_out[0, :, 0]
    pos1 = tile_start[e1] * T_ROWS + rank_out[1, :, 0]
    pos = jnp.stack([pos0, pos1]).reshape(TOPK_, N, 1)

    t_arange = jnp.arange(NT, dtype=jnp.int32)
    texp = jnp.clip(
        jnp.searchsorted(cum_tiles, t_arange, side="right"), 0, E_ - 1
    ).astype(jnp.int32)
    tvalid = (t_arange < cum_tiles[-1]).astype(jnp.int32)
    hslot = jnp.where(tvalid > 0, t_arange, NT).astype(jnp.int32)

    H2 = D_HID_ // 2

    def _seff(t, s):
        return (s + t) % 2

    h_all = pl.pallas_call(
        _up_body,
        grid_spec=pltpu.PrefetchScalarGridSpec(
            num_scalar_prefetch=3,
            grid=(NT, 2),
            in_specs=[
                pl.BlockSpec((TOPK_, N, 1), lambda t, s, texp, tv, hs: (0, 0, 0)),
                pl.BlockSpec((N, D), lambda t, s, texp, tv, hs: (0, 0)),
                pl.BlockSpec(
                    (1, D, H2), lambda t, s, texp, tv, hs: (texp[t], 0, _seff(t, s))
                ),
                pl.BlockSpec(
                    (1, 1, H2), lambda t, s, texp, tv, hs: (texp[t], 0, _seff(t, s))
                ),
            ],
            out_specs=pl.BlockSpec(
                (1, T_ROWS, H2), lambda t, s, texp, tv, hs: (hs[t], 0, _seff(t, s))
            ),
            scratch_shapes=[pltpu.VMEM((T_ROWS, D_MODEL_), jnp.float32)],
        ),
        out_shape=jax.ShapeDtypeStruct((NT + 1, T_ROWS, D_HID_), jnp.bfloat16),
        compiler_params=pltpu.CompilerParams(
            dimension_semantics=("arbitrary", "arbitrary"),
        ),
    )(texp, tvalid, hslot, pos, x_flat, w1, b1)

    out = pl.pallas_call(
        _down_body,
        grid_spec=pltpu.PrefetchScalarGridSpec(
            num_scalar_prefetch=3,
            grid=(NT, 2),
            in_specs=[
                pl.BlockSpec((TOPK_, N, 1), lambda t, s, texp, tv, hs: (0, 0, 0)),
                pl.BlockSpec((TOPK_, N, 1), lambda t, s, texp, tv, hs: (0, 0, 0)),
                pl.BlockSpec(
                    (1, T_ROWS, H2), lambda t, s, texp, tv, hs: (hs[t], 0, _seff(t, s))
                ),
                pl.BlockSpec(
                    (1, H2, D), lambda t, s, texp, tv, hs: (texp[t], _seff(t, s), 0)
                ),
                pl.BlockSpec((1, 1, D), lambda t, s, texp, tv, hs: (texp[t], 0, 0)),
            ],
            out_specs=pl.BlockSpec((N, D), lambda t, s, texp, tv, hs: (0, 0)),
            scratch_shapes=[pltpu.VMEM((T_ROWS, D_MODEL_), jnp.float32)],
        ),
        out_shape=jax.ShapeDtypeStruct((N, D), jnp.float32),
        compiler_params=pltpu.CompilerParams(
            dimension_semantics=("arbitrary", "arbitrary"),
        ),
    )(texp, tvalid, hslot, pos, w_out, h_all, w2, b2)

    return out.reshape(B, T, D)


# speedup vs baseline: 1.3831x; 1.3831x over previous
import functools

import jax
import jax.numpy as jnp
from jax.experimental import pallas as pl
from jax.experimental.pallas import tpu as pltpu

D_MODEL_ = 1024
D_HID_ = 4096
E_ = 8
TOPK_ = 2

T_ROWS = 256


def _router_body(NT, x_ref, gw_ref, pos_ref, w_ref, texp_ref, tvalid_ref,
                 hslot_ref):
    N = x_ref.shape[0]
    logits = jnp.dot(x_ref[...], gw_ref[...], preferred_element_type=jnp.float32)
    m = jnp.max(logits, axis=-1, keepdims=True)
    ex = jnp.exp(logits - m)
    probs = ex / jnp.sum(ex, axis=-1, keepdims=True)

    ncols = probs.shape[-1]
    iota = jax.lax.broadcasted_iota(jnp.int32, probs.shape, 1)
    big = jnp.int32(ncols)

    m1 = jnp.max(probs, axis=-1, keepdims=True)
    i1 = jnp.min(jnp.where(probs == m1, iota, big), axis=-1, keepdims=True)
    mask1 = iota == i1
    probs2 = jnp.where(mask1, -jnp.inf, probs)
    m2 = jnp.max(probs2, axis=-1, keepdims=True)
    i2 = jnp.min(jnp.where(probs2 == m2, iota, big), axis=-1, keepdims=True)
    mask2 = iota == i2

    denom = m1 + m2
    w_ref[0] = m1 / denom
    w_ref[1] = m2 / denom

    BK = 128
    NB = N // BK
    oh = mask1.astype(jnp.float32) + mask2.astype(jnp.float32)
    oh3 = oh.reshape(NB, BK, ncols)
    li = jax.lax.broadcasted_iota(jnp.int32, (BK, BK), 0)
    lj = jax.lax.broadcasted_iota(jnp.int32, (BK, BK), 1)
    ltri = (lj <= li).astype(jnp.float32)
    intra = [
        jnp.dot(ltri, oh3[k], preferred_element_type=jnp.float32)
        for k in range(NB)
    ]
    bs = jnp.concatenate([intra[k][BK - 1 : BK, :] for k in range(NB)], axis=0)
    si = jax.lax.broadcasted_iota(jnp.int32, (NB, NB), 0)
    sj = jax.lax.broadcasted_iota(jnp.int32, (NB, NB), 1)
    stri = (sj < si).astype(jnp.float32)
    off = jnp.dot(stri, bs, preferred_element_type=jnp.float32)
    cum_inc = jnp.concatenate(intra, axis=0) + jnp.repeat(off, BK, axis=0)
    before = cum_inc - oh
    m1f = mask1.astype(jnp.float32)
    m2f = mask2.astype(jnp.float32)
    rank0 = jnp.sum(before * m1f, axis=-1, keepdims=True)
    rank1 = jnp.sum(before * m2f, axis=-1, keepdims=True)
    counts = off[NB - 1 : NB, :] + bs[NB - 1 : NB, :]

    tf = jnp.float32(T_ROWS)
    nt_e = jnp.floor((counts + (tf - 1.0)) / tf)
    ui = jax.lax.broadcasted_iota(jnp.int32, (ncols, ncols), 0)
    uj = jax.lax.broadcasted_iota(jnp.int32, (ncols, ncols), 1)
    utri = (ui <= uj).astype(jnp.float32)
    cum_t = jnp.dot(nt_e, utri, preferred_element_type=jnp.float32)
    tile_start = cum_t - nt_e
    ts0 = jnp.sum(m1f * tile_start, axis=-1, keepdims=True)
    ts1 = jnp.sum(m2f * tile_start, axis=-1, keepdims=True)
    pos_ref[0] = (ts0 * tf + rank0).astype(jnp.int32)
    pos_ref[1] = (ts1 * tf + rank1).astype(jnp.int32)

    total_t = jnp.sum(nt_e)
    t_io = jax.lax.broadcasted_iota(jnp.int32, (NT, ncols), 0).astype(jnp.float32)
    cum_b = jnp.broadcast_to(cum_t, (NT, ncols))
    texp = jnp.minimum(
        jnp.sum((cum_b <= t_io).astype(jnp.float32), axis=-1, keepdims=True),
        jnp.float32(ncols - 1),
    )
    t_col = t_io[:, 0:1]
    tv = (t_col < total_t).astype(jnp.float32)
    texp_ref[...] = texp.astype(jnp.int32)
    tvalid_ref[...] = tv.astype(jnp.int32)
    hslot_ref[...] = jnp.where(tv > 0.0, t_col, jnp.float32(NT)).astype(jnp.int32)


def _up_body(texp_ref, tvalid_ref, hslot_ref, pos_ref, x_ref, w1_ref, b1_ref,
             h_ref):
    t = pl.program_id(0)
    N = x_ref.shape[0]

    @pl.when(tvalid_ref[t] > 0)
    def _():
        base = t * T_ROWS
        p0 = pos_ref[0, :, 0][None, :]
        p1 = pos_ref[1, :, 0][None, :]
        rowi = jax.lax.broadcasted_iota(jnp.int32, (T_ROWS, N), 0) + base
        g = ((rowi == p0) | (rowi == p1)).astype(jnp.float32)
        xs = jnp.dot(g, x_ref[...], preferred_element_type=jnp.float32)
        h = jnp.dot(xs, w1_ref[0], preferred_element_type=jnp.float32)
        h = h + b1_ref[0]
        h = 0.5 * h * (1.0 + jax.lax.erf(h * (2.0 ** -0.5)))
        h_ref[0] = h.astype(jnp.bfloat16)


def _down_body(texp_ref, tvalid_ref, hslot_ref, pos_ref, w_ref, h_ref, w2_ref,
               b2_ref, out_ref):
    t = pl.program_id(0)
    N = out_ref.shape[0]

    @pl.when(t == 0)
    def _():
        out_ref[...] = jnp.zeros_like(out_ref)

    @pl.when(tvalid_ref[t] > 0)
    def _():
        h = h_ref[0].astype(jnp.float32)
        ys = jnp.dot(h, w2_ref[0], preferred_element_type=jnp.float32)
        ys = ys + b2_ref[0]
        base = t * T_ROWS
        p0 = pos_ref[0, :, 0][:, None]
        p1 = pos_ref[1, :, 0][:, None]
        w0 = w_ref[0, :, 0][:, None]
        w1v = w_ref[1, :, 0][:, None]
        coli = jax.lax.broadcasted_iota(jnp.int32, (N, T_ROWS), 1) + base
        pw = jnp.where(coli == p0, w0, 0.0) + jnp.where(coli == p1, w1v, 0.0)
        out_ref[...] += jnp.dot(pw, ys, preferred_element_type=jnp.float32)


@jax.jit
def kernel(x, gate_w, w1, w2, b1, b2):
    B, T, D = x.shape
    N = B * T
    M = N * TOPK_
    NT = M // T_ROWS + E_
    x_flat = x.reshape(N, D)

    pos, w_out, texp2, tvalid2, hslot2 = pl.pallas_call(
        functools.partial(_router_body, NT),
        out_shape=(
            jax.ShapeDtypeStruct((TOPK_, N, 1), jnp.int32),
            jax.ShapeDtypeStruct((TOPK_, N, 1), jnp.float32),
            jax.ShapeDtypeStruct((NT, 1), jnp.int32),
            jax.ShapeDtypeStruct((NT, 1), jnp.int32),
            jax.ShapeDtypeStruct((NT, 1), jnp.int32),
        ),
    )(x_flat, gate_w)
    texp = texp2.reshape(NT)
    tvalid = tvalid2.reshape(NT)
    hslot = hslot2.reshape(NT)

    h_all = pl.pallas_call(
        _up_body,
        grid_spec=pltpu.PrefetchScalarGridSpec(
            num_scalar_prefetch=3,
            grid=(NT,),
            in_specs=[
                pl.BlockSpec((TOPK_, N, 1), lambda t, texp, tv, hs: (0, 0, 0)),
                pl.BlockSpec((N, D), lambda t, texp, tv, hs: (0, 0)),
                pl.BlockSpec((1, D, D_HID_), lambda t, texp, tv, hs: (texp[t], 0, 0)),
                pl.BlockSpec((1, 1, D_HID_), lambda t, texp, tv, hs: (texp[t], 0, 0)),
            ],
            out_specs=pl.BlockSpec(
                (1, T_ROWS, D_HID_), lambda t, texp, tv, hs: (hs[t], 0, 0)
            ),
        ),
        out_shape=jax.ShapeDtypeStruct((NT + 1, T_ROWS, D_HID_), jnp.bfloat16),
        compiler_params=pltpu.CompilerParams(
            dimension_semantics=("arbitrary",),
        ),
    )(texp, tvalid, hslot, pos, x_flat, w1, b1)

    out = pl.pallas_call(
        _down_body,
        grid_spec=pltpu.PrefetchScalarGridSpec(
            num_scalar_prefetch=3,
            grid=(NT,),
            in_specs=[
                pl.BlockSpec((TOPK_, N, 1), lambda t, texp, tv, hs: (0, 0, 0)),
                pl.BlockSpec((TOPK_, N, 1), lambda t, texp, tv, hs: (0, 0, 0)),
                pl.BlockSpec((1, T_ROWS, D_HID_), lambda t, texp, tv, hs: (hs[t], 0, 0)),
                pl.BlockSpec((1, D_HID_, D), lambda t, texp, tv, hs: (texp[t], 0, 0)),
                pl.BlockSpec((1, 1, D), lambda t, texp, tv, hs: (texp[t], 0, 0)),
            ],
            out_specs=pl.BlockSpec((N, D), lambda t, texp, tv, hs: (0, 0)),
        ),
        out_shape=jax.ShapeDtypeStruct((N, D), jnp.float32),
        compiler_params=pltpu.CompilerParams(
            dimension_semantics=("arbitrary",),
        ),
    )(texp, tvalid, hslot, pos, w_out, h_all, w2, b2)

    return out.reshape(B, T, D)
